# baseline (speedup 1.0000x reference)
import functools

import jax
import jax.numpy as jnp
from jax.experimental import pallas as pl
from jax.experimental.pallas import tpu as pltpu

ALPHA = 0.2


def _pick_block(n: int, target: int) -> int:
    b = min(target, n)
    b -= b % 8
    while b >= 8:
        if n % b == 0:
            return b
        b -= 8
    return n


def _prologue_body(x_ref, w_ref, a1_ref, a2_ref, wh_ref, c_ref, r_ref):
    wh = jnp.dot(x_ref[...], w_ref[...], preferred_element_type=jnp.float32)
    f_out = wh.shape[1]
    s_dst = jnp.dot(wh, a2_ref[...], preferred_element_type=jnp.float32)
    s_src = jnp.dot(wh, a1_ref[...], preferred_element_type=jnp.float32)
    v2 = jnp.exp(-ALPHA * s_dst)
    c_ref[...] = jnp.exp(-(1.0 - ALPHA) * s_dst)
    r_ref[...] = jnp.exp((1.0 - ALPHA) * s_src)
    lane = jax.lax.broadcasted_iota(jnp.int32, (wh.shape[0], 8), 1)
    wh_ref[:, :f_out] = v2 * wh
    wh_ref[:, f_out:] = jnp.where(lane == 0, v2, 0.0)


def _main_body(nj, bm, bn, f_out, adj_ref, wh_ref, c_ref, r_ref,
               out_ref, acc_ref):
    j = pl.program_id(1)

    @pl.when(j == 0)
    def _init():
        acc_ref[...] = jnp.zeros_like(acc_ref)

    e = adj_ref[...] * jnp.minimum(c_ref[...], r_ref[...])
    wh_j = wh_ref[pl.ds(j * bn, bn), :]
    acc_ref[...] += jnp.dot(e, wh_j, preferred_element_type=jnp.float32)

    @pl.when(j == nj - 1)
    def _finish():
        h = acc_ref[:, :f_out] / acc_ref[:, f_out:f_out + 1]
        out_ref[...] = jnp.where(h > 0, h, jnp.exp(jnp.minimum(h, 0.0)) - 1.0)


def kernel(input, adj, W, a):
    n, f_in = input.shape
    f_out = W.shape[1]
    a1 = a[0, :f_out].reshape(f_out, 1)
    a2 = a[0, f_out:].reshape(f_out, 1)

    bp = _pick_block(n, 2000)
    np_ = n // bp
    wh, cc, rc = pl.pallas_call(
        _prologue_body,
        grid=(np_,),
        in_specs=[
            pl.BlockSpec((bp, f_in), lambda i: (i, 0)),
            pl.BlockSpec((f_in, f_out), lambda i: (0, 0)),
            pl.BlockSpec((f_out, 1), lambda i: (0, 0)),
            pl.BlockSpec((f_out, 1), lambda i: (0, 0)),
        ],
        out_specs=[
            pl.BlockSpec((bp, f_out + 8), lambda i: (i, 0)),
            pl.BlockSpec((bp, 1), lambda i: (i, 0)),
            pl.BlockSpec((bp, 1), lambda i: (i, 0)),
        ],
        out_shape=[
            jax.ShapeDtypeStruct((n, f_out + 8), jnp.float32),
            jax.ShapeDtypeStruct((n, 1), jnp.float32),
            jax.ShapeDtypeStruct((n, 1), jnp.float32),
        ],
    )(input, W, a1, a2)

    c = cc.reshape(1, n)

    bm = _pick_block(n, 200)
    bn = n
    ni, nj = n // bm, n // bn
    out = pl.pallas_call(
        functools.partial(_main_body, nj, bm, bn, f_out),
        grid=(ni, nj),
        in_specs=[
            pl.BlockSpec((bm, bn), lambda i, j: (i, j)),
            pl.BlockSpec((n, f_out + 8), lambda i, j: (0, 0)),
            pl.BlockSpec((1, bn), lambda i, j: (0, j)),
            pl.BlockSpec((bm, 1), lambda i, j: (i, 0)),
        ],
        out_specs=pl.BlockSpec((bm, f_out), lambda i, j: (i, 0)),
        out_shape=jax.ShapeDtypeStruct((n, f_out), jnp.float32),
        scratch_shapes=[
            pltpu.VMEM((bm, f_out + 8), jnp.float32),
        ],
        compiler_params=pltpu.CompilerParams(
            dimension_semantics=("parallel", "arbitrary")),
    )(adj, wh, c, rc)
    return out
